# baseline (speedup 1.0000x reference)
import functools

import jax
import jax.numpy as jnp
from jax import lax
from jax.experimental import pallas as pl
from jax.experimental.pallas import tpu as pltpu
from jax.experimental.pallas import tpu_sc as plsc

N = 10000
E = 320000
D = 128

NC = 2
NS = 16
NW = NC * NS


def _lorentz_post(h, es):
    time = (1.0 / (1.0 + jnp.exp(-h[:, :1]))) * es + 1.1
    hsq = h * h
    sq = jnp.sum(hsq, axis=1, keepdims=True) - hsq[:, :1]
    sq = jnp.clip(sq, 1e-8, None)
    scale = (time * time - 1.0) / sq
    root = jnp.sqrt(scale)
    col = lax.broadcasted_iota(jnp.int32, h.shape, 1)
    return jnp.where(col == 0, time, h * root)


def _lnormalize(z):
    zsq = z * z
    negl = 2.0 * zsq[:, :1] - jnp.sum(zsq, axis=1, keepdims=True)
    denom = jnp.sqrt(jnp.clip(jnp.abs(negl), 1e-8, None))
    return z / denom


def _tc_embed1_body(x_ref, w1a_ref, w1bt_ref, b1_ref, s1_ref, out_ref):
    x = x_ref[...]
    sq = jnp.sum(x * x, axis=1, keepdims=True)
    nrm = jnp.sqrt(jnp.clip(sq, 1e-8, None))
    en = jnp.exp(nrm)
    eni = 1.0 / en
    csh = 0.5 * (en + eni)
    snh = 0.5 * (en - eni)
    sp = x * (snh / nrm)
    h = jnp.dot(sp, w1bt_ref[...], preferred_element_type=jnp.float32)
    h = h + csh * w1a_ref[...] + b1_ref[...]
    es = jnp.exp(s1_ref[...])
    out_ref[...] = _lorentz_post(h, es)


def _tc_mid_body(resnet, p0_ref, p1_ref, prev_ref, wt_ref, b_ref, s_ref,
                 x_out_ref, h_out_ref):
    sup = p0_ref[...] + p1_ref[...]
    agg = _lnormalize(sup)
    if resnet:
        xi = _lnormalize(prev_ref[...] + agg)
    else:
        xi = agg
    x_out_ref[...] = xi
    hin = jnp.maximum(xi, 0.0)
    h = jnp.dot(hin, wt_ref[...], preferred_element_type=jnp.float32)
    h = h + b_ref[...]
    es = jnp.exp(s_ref[...])
    h_out_ref[...] = _lorentz_post(h, es)


def _tc_final_body(p0_ref, p1_ref, x3_ref, out_ref):
    x4 = _lnormalize(p0_ref[...] + p1_ref[...])
    out_ref[...] = _lnormalize(x3_ref[...] + x4)


def _tc_embed1(x, w1a, w1bt, b1, s1):
    return pl.pallas_call(
        _tc_embed1_body,
        out_shape=jax.ShapeDtypeStruct((N, D), jnp.float32),
    )(x, w1a, w1bt, b1, s1)


def _tc_mid(p, prev, wt, b, s, resnet):
    return pl.pallas_call(
        functools.partial(_tc_mid_body, resnet),
        out_shape=(
            jax.ShapeDtypeStruct((N, D), jnp.float32),
            jax.ShapeDtypeStruct((N, D), jnp.float32),
        ),
    )(p[0, :N], p[1, :N], prev, wt, b, s)


def _tc_final(p, x3):
    return pl.pallas_call(
        _tc_final_body,
        out_shape=jax.ShapeDtypeStruct((N, D), jnp.float32),
    )(p[0, :N], p[1, :N], x3)


C0 = 112
C1 = 48
CMAX = max(C0, C1)
CHUNK = 128
TOTAL_CHUNKS = NS * (C0 + C1)
TOTAL_STAGE = TOTAL_CHUNKS + CMAX
EPAD = TOTAL_CHUNKS * CHUNK
NPAD = 10240
ROWS_PER_TILE = NPAD // NS


def _sc_agg_body(h_hbm, src_hbm, dst_hbm, w_hbm, out_hbm,
                 src_v, dst_v, rows_v, w8a, w8b, acc,
                 gsem, wsema, wsemb):
    w8 = (w8a, w8b)
    wsems = (wsema, wsemb)
    c = lax.axis_index("c")
    s = lax.axis_index("s")
    cbase = jnp.where(c == 0, s * C0, NS * C0 + s * C1)
    cbase = pl.multiple_of(cbase, 16)
    cc = jnp.where(c == 0, C0, C1)

    def zrow(r, carry):
        for j in range(D // 16):
            rows_v[r, pl.ds(16 * j, 16)] = jnp.zeros((16,), jnp.float32)
        return carry

    lax.fori_loop(0, CHUNK, zrow, 0)
    for q in range(ROWS_PER_TILE // CHUNK):
        pltpu.sync_copy(
            rows_v, acc.at[pl.ds(s * ROWS_PER_TILE + q * CHUNK, CHUNK)])
    pltpu.sync_copy(src_hbm.at[pl.ds(cbase, CMAX)], src_v)
    pltpu.sync_copy(dst_hbm.at[pl.ds(cbase, CMAX)], dst_v)
    plsc.subcore_barrier()

    def start_w8(g, p):
        pltpu.async_copy(w_hbm.at[pl.ds(cbase + g * 8, 8)], w8[p], wsems[p])

    def wait_w8(g, p):
        pltpu.make_async_copy(w_hbm.at[pl.ds(cbase + g * 8, 8)], w8[p],
                              wsems[p]).wait()

    ngroups = cc // 8

    def turn(i, k, p):
        pltpu.async_copy(h_hbm.at[src_v.at[i]], rows_v, gsem).wait()

        def edge_body(e, carry2):
            wv = plsc.load_gather(
                w8[p], [jnp.full((16,), k, jnp.int32),
                        jnp.full((16,), e, jnp.int32)])
            for j in range(D // 16):
                rows_v[e, pl.ds(16 * j, 16)] = (
                    rows_v[e, pl.ds(16 * j, 16)] * wv)
            return carry2

        lax.fori_loop(0, CHUNK, edge_body, 0, unroll=4)
        pltpu.sync_copy(rows_v, acc.at[dst_v.at[i]], add=True)

    start_w8(0, 0)
    start_w8(1, 1)

    def pair_body(g2, carry):
        for p in range(2):
            g = g2 * 2 + p
            wait_w8(g, p)
            for k in range(8):
                turn(g * 8 + k, k, p)

            @pl.when(g + 2 < ngroups)
            def _():
                start_w8(g + 2, p)

        return carry

    lax.fori_loop(0, cc // 16, pair_body, 0)

    plsc.subcore_barrier()
    pltpu.sync_copy(acc.at[pl.ds(s * ROWS_PER_TILE, ROWS_PER_TILE)],
                    out_hbm.at[c, pl.ds(s * ROWS_PER_TILE, ROWS_PER_TILE)])


def _make_sc_agg():
    return pl.kernel(
        _sc_agg_body,
        mesh=plsc.VectorSubcoreMesh(core_axis_name="c", subcore_axis_name="s"),
        compiler_params=pltpu.CompilerParams(needs_layout_passes=False),
        out_type=jax.ShapeDtypeStruct((NC, NPAD, D), jnp.float32),
        scratch_types=(
            [pltpu.VMEM((CMAX, CHUNK), jnp.int32),
             pltpu.VMEM((CMAX, CHUNK), jnp.int32),
             pltpu.VMEM((CHUNK, D), jnp.float32),
             pltpu.VMEM((8, CHUNK), jnp.float32),
             pltpu.VMEM((8, CHUNK), jnp.float32),
             pltpu.VMEM_SHARED((NPAD, D), jnp.float32)]
            + [pltpu.SemaphoreType.DMA for _ in range(3)]
        ),
    )


def kernel(x, edge_index, edge_weight, W1, b1, s1, W2, b2, s2, W3, b3, s3,
           W4, b4, s4):
    npad_e = TOTAL_STAGE * CHUNK - E
    src = jnp.concatenate([edge_index[0], jnp.zeros((npad_e,), jnp.int32)])
    dst = jnp.concatenate([edge_index[1], jnp.zeros((npad_e,), jnp.int32)])
    w = jnp.concatenate([edge_weight, jnp.zeros((npad_e,), jnp.float32)])
    src = src.reshape(TOTAL_STAGE, CHUNK)
    dst = dst.reshape(TOTAL_STAGE, CHUNK)
    w = w.reshape(TOTAL_STAGE, CHUNK)

    w1a = W1[:, 0].reshape(1, D)
    w1bt = W1[:, 1:].T
    b1r = b1.reshape(1, D)
    s1r = s1.reshape(1, 1)

    agg = _make_sc_agg()

    h1 = _tc_embed1(x, w1a, w1bt, b1r, s1r)
    p1 = agg(h1, src, dst, w)
    x1, h2 = _tc_mid(p1, h1, W2.T, b2.reshape(1, D), s2.reshape(1, 1),
                     resnet=False)
    p2 = agg(h2, src, dst, w)
    x2, h3 = _tc_mid(p2, x1, W3.T, b3.reshape(1, D), s3.reshape(1, 1),
                     resnet=True)
    p3 = agg(h3, src, dst, w)
    x3, h4 = _tc_mid(p3, x2, W4.T, b4.reshape(1, D), s4.reshape(1, 1),
                     resnet=True)
    p4 = agg(h4, src, dst, w)
    return _tc_final(p4, x3)


# speedup vs baseline: 1.0342x; 1.0342x over previous
import functools

import jax
import jax.numpy as jnp
from jax import lax
from jax.experimental import pallas as pl
from jax.experimental.pallas import tpu as pltpu
from jax.experimental.pallas import tpu_sc as plsc

N = 10000
E = 320000
D = 128

NC = 2
NS = 16
NW = NC * NS


def _lorentz_post(h, es):
    time = (1.0 / (1.0 + jnp.exp(-h[:, :1]))) * es + 1.1
    hsq = h * h
    sq = jnp.sum(hsq, axis=1, keepdims=True) - hsq[:, :1]
    sq = jnp.clip(sq, 1e-8, None)
    scale = (time * time - 1.0) / sq
    root = jnp.sqrt(scale)
    col = lax.broadcasted_iota(jnp.int32, h.shape, 1)
    return jnp.where(col == 0, time, h * root)


def _lnormalize(z):
    zsq = z * z
    negl = 2.0 * zsq[:, :1] - jnp.sum(zsq, axis=1, keepdims=True)
    denom = jnp.sqrt(jnp.clip(jnp.abs(negl), 1e-8, None))
    return z / denom


def _tc_embed1_body(x_ref, w1a_ref, w1bt_ref, b1_ref, s1_ref, out_ref):
    x = x_ref[...]
    sq = jnp.sum(x * x, axis=1, keepdims=True)
    nrm = jnp.sqrt(jnp.clip(sq, 1e-8, None))
    en = jnp.exp(nrm)
    eni = 1.0 / en
    csh = 0.5 * (en + eni)
    snh = 0.5 * (en - eni)
    sp = x * (snh / nrm)
    h = jnp.dot(sp, w1bt_ref[...], preferred_element_type=jnp.float32)
    h = h + csh * w1a_ref[...] + b1_ref[...]
    es = jnp.exp(s1_ref[...])
    out_ref[...] = _lorentz_post(h, es)


def _tc_mid_body(resnet, p0_ref, p1_ref, prev_ref, wt_ref, b_ref, s_ref,
                 x_out_ref, h_out_ref):
    sup = p0_ref[...] + p1_ref[...]
    agg = _lnormalize(sup)
    if resnet:
        xi = _lnormalize(prev_ref[...] + agg)
    else:
        xi = agg
    x_out_ref[...] = xi
    hin = jnp.maximum(xi, 0.0)
    h = jnp.dot(hin, wt_ref[...], preferred_element_type=jnp.float32)
    h = h + b_ref[...]
    es = jnp.exp(s_ref[...])
    h_out_ref[...] = _lorentz_post(h, es)


def _tc_final_body(p0_ref, p1_ref, x3_ref, out_ref):
    x4 = _lnormalize(p0_ref[...] + p1_ref[...])
    out_ref[...] = _lnormalize(x3_ref[...] + x4)


def _tc_embed1(x, w1a, w1bt, b1, s1):
    return pl.pallas_call(
        _tc_embed1_body,
        out_shape=jax.ShapeDtypeStruct((N, D), jnp.float32),
    )(x, w1a, w1bt, b1, s1)


def _tc_mid(p, prev, wt, b, s, resnet):
    return pl.pallas_call(
        functools.partial(_tc_mid_body, resnet),
        out_shape=(
            jax.ShapeDtypeStruct((N, D), jnp.float32),
            jax.ShapeDtypeStruct((N, D), jnp.float32),
        ),
    )(p[0, :N], p[1, :N], prev, wt, b, s)


def _tc_final(p, x3):
    return pl.pallas_call(
        _tc_final_body,
        out_shape=jax.ShapeDtypeStruct((N, D), jnp.float32),
    )(p[0, :N], p[1, :N], x3)


C0 = 112
C1 = 48
CMAX = max(C0, C1)
CHUNK = 128
TOTAL_CHUNKS = NS * (C0 + C1)
TOTAL_STAGE = TOTAL_CHUNKS + CMAX
EPAD = TOTAL_CHUNKS * CHUNK
NPAD = 10240
ROWS_PER_TILE = NPAD // NS


def _sc_agg_body(h_hbm, src_hbm, dst_hbm, w_hbm, zeros_hbm, out_hbm,
                 src_v, dst_v, rows_v, w8a, w8b, acc,
                 gsem, wsema, wsemb):
    w8 = (w8a, w8b)
    wsems = (wsema, wsemb)
    c = lax.axis_index("c")
    s = lax.axis_index("s")
    cbase = jnp.where(c == 0, s * C0, NS * C0 + s * C1)
    cbase = pl.multiple_of(cbase, 16)
    cc = jnp.where(c == 0, C0, C1)

    pltpu.sync_copy(zeros_hbm.at[pl.ds(s * ROWS_PER_TILE, ROWS_PER_TILE)],
                    acc.at[pl.ds(s * ROWS_PER_TILE, ROWS_PER_TILE)])
    pltpu.sync_copy(src_hbm.at[pl.ds(cbase, CMAX)], src_v)
    pltpu.sync_copy(dst_hbm.at[pl.ds(cbase, CMAX)], dst_v)
    plsc.subcore_barrier()

    def start_w8(g, p):
        pltpu.async_copy(w_hbm.at[pl.ds(cbase + g * 8, 8)], w8[p], wsems[p])

    def wait_w8(g, p):
        pltpu.make_async_copy(w_hbm.at[pl.ds(cbase + g * 8, 8)], w8[p],
                              wsems[p]).wait()

    ngroups = cc // 8

    def turn(i, k, p):
        pltpu.async_copy(h_hbm.at[src_v.at[i]], rows_v, gsem).wait()

        def edge_body(e, carry2):
            wv = plsc.load_gather(
                w8[p], [jnp.full((16,), k, jnp.int32),
                        jnp.full((16,), e, jnp.int32)])
            for j in range(D // 16):
                rows_v[e, pl.ds(16 * j, 16)] = (
                    rows_v[e, pl.ds(16 * j, 16)] * wv)
            return carry2

        lax.fori_loop(0, CHUNK, edge_body, 0, unroll=4)
        pltpu.sync_copy(rows_v, acc.at[dst_v.at[i]], add=True)

    start_w8(0, 0)
    start_w8(1, 1)

    def pair_body(g2, carry):
        for p in range(2):
            g = g2 * 2 + p
            wait_w8(g, p)
            for k in range(8):
                turn(g * 8 + k, k, p)

            @pl.when(g + 2 < ngroups)
            def _():
                start_w8(g + 2, p)

        return carry

    lax.fori_loop(0, cc // 16, pair_body, 0)

    plsc.subcore_barrier()
    pltpu.sync_copy(acc.at[pl.ds(s * ROWS_PER_TILE, ROWS_PER_TILE)],
                    out_hbm.at[c, pl.ds(s * ROWS_PER_TILE, ROWS_PER_TILE)])


def _make_sc_agg():
    return pl.kernel(
        _sc_agg_body,
        mesh=plsc.VectorSubcoreMesh(core_axis_name="c", subcore_axis_name="s"),
        compiler_params=pltpu.CompilerParams(needs_layout_passes=False),
        out_type=jax.ShapeDtypeStruct((NC, NPAD, D), jnp.float32),
        scratch_types=(
            [pltpu.VMEM((CMAX, CHUNK), jnp.int32),
             pltpu.VMEM((CMAX, CHUNK), jnp.int32),
             pltpu.VMEM((CHUNK, D), jnp.float32),
             pltpu.VMEM((8, CHUNK), jnp.float32),
             pltpu.VMEM((8, CHUNK), jnp.float32),
             pltpu.VMEM_SHARED((NPAD, D), jnp.float32)]
            + [pltpu.SemaphoreType.DMA for _ in range(3)]
        ),
    )


def kernel(x, edge_index, edge_weight, W1, b1, s1, W2, b2, s2, W3, b3, s3,
           W4, b4, s4):
    npad_e = TOTAL_STAGE * CHUNK - E
    src = jnp.concatenate([edge_index[0], jnp.zeros((npad_e,), jnp.int32)])
    dst = jnp.concatenate([edge_index[1], jnp.zeros((npad_e,), jnp.int32)])
    w = jnp.concatenate([edge_weight, jnp.zeros((npad_e,), jnp.float32)])
    src = src.reshape(TOTAL_STAGE, CHUNK)
    dst = dst.reshape(TOTAL_STAGE, CHUNK)
    w = w.reshape(TOTAL_STAGE, CHUNK)
    zeros = jnp.zeros((NPAD, D), jnp.float32)

    w1a = W1[:, 0].reshape(1, D)
    w1bt = W1[:, 1:].T
    b1r = b1.reshape(1, D)
    s1r = s1.reshape(1, 1)

    agg = _make_sc_agg()

    h1 = _tc_embed1(x, w1a, w1bt, b1r, s1r)
    p1 = agg(h1, src, dst, w, zeros)
    x1, h2 = _tc_mid(p1, h1, W2.T, b2.reshape(1, D), s2.reshape(1, 1),
                     resnet=False)
    p2 = agg(h2, src, dst, w, zeros)
    x2, h3 = _tc_mid(p2, x1, W3.T, b3.reshape(1, D), s3.reshape(1, 1),
                     resnet=True)
    p3 = agg(h3, src, dst, w, zeros)
    x3, h4 = _tc_mid(p3, x2, W4.T, b4.reshape(1, D), s4.reshape(1, 1),
                     resnet=True)
    p4 = agg(h4, src, dst, w, zeros)
    return _tc_final(p4, x3)
